# baseline (speedup 1.0000x reference)
import functools

import jax
import jax.numpy as jnp
from jax import lax
from jax.experimental import pallas as pl
from jax.experimental.pallas import tpu as pltpu
from jax.experimental.pallas import tpu_sc as plsc

_VOCAB = 100000
_D = 64
_B = 16384
_CTX = 20
_NEG = 20

_NC = 2
_NS = 16
_NW = _NC * _NS
_PW = _B // _NW

_CH = 16
_NCHUNK = _PW // _CH
_IDXBLK = 80
_NBLK = (_CH * _CTX) // _IDXBLK
_WBLK = (_PW * _CTX) // _IDXBLK
_COLS = 32
_NBUF = 2


def _sc_body(ctx_hbm, tgt_hbm, neg_hbm, win_hbm, wout_hbm, out_hbm,
             ctx_idxw, neg_idxw, tgt_idxw,
             ctx_rows, neg_rows, tgt_rows, scores, sems):
    cid = lax.axis_index("c")
    sid = lax.axis_index("s")
    wid = sid * _NC + cid
    base = wid * _PW
    iw0 = pl.multiple_of(wid * _WBLK, 8)

    pltpu.sync_copy(ctx_hbm.at[pl.ds(iw0, _WBLK)], ctx_idxw)
    pltpu.sync_copy(neg_hbm.at[pl.ds(iw0, _WBLK)], neg_idxw)
    pltpu.sync_copy(tgt_hbm.at[pl.ds(pl.multiple_of(base, _PW), _PW)], tgt_idxw)

    def fire(ci, b):
        for k in range(_NBLK):
            pltpu.async_copy(
                win_hbm.at[ctx_idxw.at[ci * _NBLK + k]],
                ctx_rows[b].at[pl.ds(k * _IDXBLK, _IDXBLK)], sems[b])
        for k in range(_NBLK):
            pltpu.async_copy(
                wout_hbm.at[neg_idxw.at[ci * _NBLK + k]],
                neg_rows[b].at[pl.ds(k * _IDXBLK, _IDXBLK)], sems[b])
        pltpu.async_copy(
            wout_hbm.at[tgt_idxw.at[pl.ds(pl.multiple_of(ci * _CH, _CH), _CH)]],
            tgt_rows[b], sems[b])

    def drain(b):
        for k in range(_NBLK):
            pltpu.make_async_copy(
                win_hbm.at[ctx_idxw.at[0]],
                ctx_rows[b].at[pl.ds(k * _IDXBLK, _IDXBLK)], sems[b]).wait()
            pltpu.make_async_copy(
                wout_hbm.at[neg_idxw.at[0]],
                neg_rows[b].at[pl.ds(k * _IDXBLK, _IDXBLK)], sems[b]).wait()
        pltpu.make_async_copy(
            wout_hbm.at[tgt_idxw.at[pl.ds(0, _CH)]],
            tgt_rows[b], sems[b]).wait()

    lanes = lax.iota(jnp.int32, 16)

    def hsum(v):
        return jnp.sum(v)

    def compute(ci, b):
        crows, nrows, trows, sc_out = ctx_rows[b], neg_rows[b], tgt_rows[b], scores[b]

        def row_body(r, c2):
            rb = r * _CTX
            hs = []
            for q in range(4):
                acc = crows[rb, pl.ds(q * 16, 16)]
                for j in range(1, _CTX):
                    acc = acc + crows[rb + j, pl.ds(q * 16, 16)]
                hs.append(acc * (1.0 / _CTX))
            p = hs[0] * trows[r, pl.ds(0, 16)]
            for q in range(1, 4):
                p = p + hs[q] * trows[r, pl.ds(q * 16, 16)]
            acc0 = jnp.where(lanes == 0, hsum(p), jnp.zeros((16,), jnp.float32))
            acc1 = jnp.zeros((16,), jnp.float32)
            nb = r * _NEG
            for n in range(_NEG):
                s = hs[0] * nrows[nb + n, pl.ds(0, 16)]
                for q in range(1, 4):
                    s = s + hs[q] * nrows[nb + n, pl.ds(q * 16, 16)]
                col = 1 + n
                if col < 16:
                    acc0 = jnp.where(lanes == col, hsum(s), acc0)
                else:
                    acc1 = jnp.where(lanes == col - 16, hsum(s), acc1)
            sc_out[r, pl.ds(0, 16)] = acc0
            sc_out[r, pl.ds(16, 16)] = acc1
            return c2

        lax.fori_loop(0, _CH, row_body, 0, unroll=4)
        row0 = pl.multiple_of(base + ci * _CH, _CH)
        pltpu.sync_copy(sc_out, out_hbm.at[pl.ds(row0, _CH)])

    for b in range(_NBUF):
        fire(b, b)

    def loop_body(g):
        for b in range(_NBUF):
            ci = g + b
            drain(b)

            @pl.when(ci + _NBUF < _NCHUNK)
            def _():
                fire(ci + _NBUF, b)

            compute(ci, b)

    pl.loop(0, _NCHUNK, step=_NBUF)(loop_body)


_sc_scores = pl.kernel(
    _sc_body,
    out_type=jax.ShapeDtypeStruct((_B, _COLS), jnp.float32),
    mesh=plsc.VectorSubcoreMesh(core_axis_name="c", subcore_axis_name="s"),
    compiler_params=pltpu.CompilerParams(
        use_tc_tiling_on_sc=False, needs_layout_passes=False),
    scratch_types=[
        pltpu.VMEM((_WBLK, _IDXBLK), jnp.int32),
        pltpu.VMEM((_WBLK, _IDXBLK), jnp.int32),
        pltpu.VMEM((_PW,), jnp.int32),
        [pltpu.VMEM((_CH * _CTX, _D), jnp.float32)] * _NBUF,
        [pltpu.VMEM((_CH * _NEG, _D), jnp.float32)] * _NBUF,
        [pltpu.VMEM((_CH, _D), jnp.float32)] * _NBUF,
        [pltpu.VMEM((_CH, _COLS), jnp.float32)] * _NBUF,
        [pltpu.SemaphoreType.DMA] * _NBUF,
    ],
)


def _tc_loss_body(s_ref, o_ref):
    x = s_ref[...]
    col = lax.broadcasted_iota(jnp.int32, x.shape, 1) % _COLS
    y = jnp.where(col == 0, -x, x)
    sp = jnp.maximum(y, 0.0) + jnp.log1p(jnp.exp(-jnp.abs(y)))
    z = jnp.where(col < 1 + _NEG, sp, 0.0)
    o_ref[...] = jnp.sum(z, keepdims=True) * (1.0 / _B)


_tc_loss = pl.pallas_call(
    _tc_loss_body,
    out_shape=jax.ShapeDtypeStruct((1, 1), jnp.float32),
)


def kernel(context, target, neg_samples, W_in, W_out):
    ctx_blk = context.reshape(-1, _IDXBLK)
    neg_blk = neg_samples.reshape(-1, _IDXBLK)
    scores = _sc_scores(ctx_blk, target, neg_blk, W_in, W_out)
    loss = _tc_loss(scores.reshape(_B * _COLS // 1024, 1024))
    return loss[0, 0]


# speedup vs baseline: 1.0648x; 1.0648x over previous
import functools

import jax
import jax.numpy as jnp
from jax import lax
from jax.experimental import pallas as pl
from jax.experimental.pallas import tpu as pltpu
from jax.experimental.pallas import tpu_sc as plsc

_VOCAB = 100000
_D = 64
_B = 16384
_CTX = 20
_NEG = 20

_NC = 2
_NS = 16
_NW = _NC * _NS
_PW = _B // _NW

_CH = 16
_NCHUNK = _PW // _CH
_IDXBLK = 80
_NBLK = (_CH * _CTX) // _IDXBLK
_WBLK = (_PW * _CTX) // _IDXBLK
_COLS = 32
_NBUF = 2


def _sc_body(ctx_hbm, tgt_hbm, neg_hbm, win_hbm, wout_hbm, out_hbm,
             ctx_idxw, neg_idxw, tgt_idxw,
             ctx_rows, neg_rows, tgt_rows, scores, sems):
    cid = lax.axis_index("c")
    sid = lax.axis_index("s")
    wid = sid * _NC + cid
    base = wid * _PW
    iw0 = pl.multiple_of(wid * _WBLK, 8)

    pltpu.sync_copy(ctx_hbm.at[pl.ds(iw0, _WBLK)], ctx_idxw)
    pltpu.sync_copy(neg_hbm.at[pl.ds(iw0, _WBLK)], neg_idxw)
    pltpu.sync_copy(tgt_hbm.at[pl.ds(pl.multiple_of(base, _PW), _PW)], tgt_idxw)

    def fire(ci, b):
        for k in range(_NBLK):
            pltpu.async_copy(
                win_hbm.at[ctx_idxw.at[ci * _NBLK + k]],
                ctx_rows[b].at[pl.ds(k * _IDXBLK, _IDXBLK)], sems[b])
        for k in range(_NBLK):
            pltpu.async_copy(
                wout_hbm.at[neg_idxw.at[ci * _NBLK + k]],
                neg_rows[b].at[pl.ds(k * _IDXBLK, _IDXBLK)], sems[b])
        pltpu.async_copy(
            wout_hbm.at[tgt_idxw.at[pl.ds(pl.multiple_of(ci * _CH, _CH), _CH)]],
            tgt_rows[b], sems[b])

    def drain(b):
        for k in range(_NBLK):
            pltpu.make_async_copy(
                win_hbm.at[ctx_idxw.at[0]],
                ctx_rows[b].at[pl.ds(k * _IDXBLK, _IDXBLK)], sems[b]).wait()
            pltpu.make_async_copy(
                wout_hbm.at[neg_idxw.at[0]],
                neg_rows[b].at[pl.ds(k * _IDXBLK, _IDXBLK)], sems[b]).wait()
        pltpu.make_async_copy(
            wout_hbm.at[tgt_idxw.at[pl.ds(0, _CH)]],
            tgt_rows[b], sems[b]).wait()

    lanes = lax.iota(jnp.int32, 16)

    def hsum(v):
        return jnp.sum(v)

    def compute(ci, b):
        crows, nrows, trows, sc_out = ctx_rows[b], neg_rows[b], tgt_rows[b], scores[b]

        def row_body(r, c2):
            rb = r * _CTX
            hs = []
            for q in range(4):
                acc = crows[rb, pl.ds(q * 16, 16)]
                for j in range(1, _CTX):
                    acc = acc + crows[rb + j, pl.ds(q * 16, 16)]
                hs.append(acc * (1.0 / _CTX))
            p = hs[0] * trows[r, pl.ds(0, 16)]
            for q in range(1, 4):
                p = p + hs[q] * trows[r, pl.ds(q * 16, 16)]
            acc0 = jnp.where(lanes == 0, hsum(p), jnp.zeros((16,), jnp.float32))
            acc1 = jnp.zeros((16,), jnp.float32)
            nb = r * _NEG
            for n in range(_NEG):
                s = hs[0] * nrows[nb + n, pl.ds(0, 16)]
                for q in range(1, 4):
                    s = s + hs[q] * nrows[nb + n, pl.ds(q * 16, 16)]
                col = 1 + n
                if col < 16:
                    acc0 = jnp.where(lanes == col, hsum(s), acc0)
                else:
                    acc1 = jnp.where(lanes == col - 16, hsum(s), acc1)
            sc_out[r, pl.ds(0, 16)] = acc0
            sc_out[r, pl.ds(16, 16)] = acc1
            return c2

        lax.fori_loop(0, _CH, row_body, 0, unroll=2)
        row0 = pl.multiple_of(base + ci * _CH, _CH)
        pltpu.sync_copy(sc_out, out_hbm.at[pl.ds(row0, _CH)])

    for b in range(_NBUF):
        fire(b, b)

    def loop_body(g):
        for b in range(_NBUF):
            ci = g + b
            drain(b)

            @pl.when(ci + _NBUF < _NCHUNK)
            def _():
                fire(ci + _NBUF, b)

            compute(ci, b)

    pl.loop(0, _NCHUNK, step=_NBUF)(loop_body)


_sc_scores = pl.kernel(
    _sc_body,
    out_type=jax.ShapeDtypeStruct((_B, _COLS), jnp.float32),
    mesh=plsc.VectorSubcoreMesh(core_axis_name="c", subcore_axis_name="s"),
    compiler_params=pltpu.CompilerParams(
        use_tc_tiling_on_sc=False, needs_layout_passes=False),
    scratch_types=[
        pltpu.VMEM((_WBLK, _IDXBLK), jnp.int32),
        pltpu.VMEM((_WBLK, _IDXBLK), jnp.int32),
        pltpu.VMEM((_PW,), jnp.int32),
        [pltpu.VMEM((_CH * _CTX, _D), jnp.float32)] * _NBUF,
        [pltpu.VMEM((_CH * _NEG, _D), jnp.float32)] * _NBUF,
        [pltpu.VMEM((_CH, _D), jnp.float32)] * _NBUF,
        [pltpu.VMEM((_CH, _COLS), jnp.float32)] * _NBUF,
        [pltpu.SemaphoreType.DMA] * _NBUF,
    ],
)


def _tc_loss_body(s_ref, o_ref):
    x = s_ref[...]
    col = lax.broadcasted_iota(jnp.int32, x.shape, 1) % _COLS
    y = jnp.where(col == 0, -x, x)
    sp = jnp.maximum(y, 0.0) + jnp.log1p(jnp.exp(-jnp.abs(y)))
    z = jnp.where(col < 1 + _NEG, sp, 0.0)
    o_ref[...] = jnp.sum(z, keepdims=True) * (1.0 / _B)


_tc_loss = pl.pallas_call(
    _tc_loss_body,
    out_shape=jax.ShapeDtypeStruct((1, 1), jnp.float32),
)


def kernel(context, target, neg_samples, W_in, W_out):
    ctx_blk = context.reshape(-1, _IDXBLK)
    neg_blk = neg_samples.reshape(-1, _IDXBLK)
    scores = _sc_scores(ctx_blk, target, neg_blk, W_in, W_out)
    loss = _tc_loss(scores.reshape(_B * _COLS // 1024, 1024))
    return loss[0, 0]
